# baseline (speedup 1.0000x reference)
import functools

import jax
import jax.numpy as jnp
from jax import lax
from jax.experimental import pallas as pl
from jax.experimental.pallas import tpu as pltpu
from jax.experimental.pallas import tpu_sc as plsc

VOCAB = 1002
N_D = 64
BATCH = 4096
HIST = 200

NW = 32
RPW = BATCH // NW
CH0 = 128
CH1 = HIST - CH0

NBUF = 4
NG = RPW // NBUF

_mesh = plsc.VectorSubcoreMesh(core_axis_name="c", subcore_axis_name="s")


@functools.partial(
    pl.kernel,
    mesh=_mesh,
    out_type=jax.ShapeDtypeStruct((BATCH, HIST, N_D), jnp.float32),
    scratch_types=[
        pltpu.VMEM((RPW, HIST), jnp.int32),
        pltpu.VMEM((NBUF, HIST, N_D), jnp.float32),
        pltpu.SemaphoreType.DMA((NBUF,)),
        pltpu.SemaphoreType.DMA((NBUF,)),
    ],
    compiler_params=pltpu.CompilerParams(use_tc_tiling_on_sc=False),
)
def _sc_embed(idx_hbm, table_hbm, out_hbm, idx_v, rows_v, gsem, ssem):
    c = lax.axis_index("c")
    s = lax.axis_index("s")
    wid = s * 2 + c
    row_base = wid * RPW
    pltpu.sync_copy(idx_hbm.at[pl.ds(row_base, RPW)], idx_v)

    def gather(i, b):
        pltpu.async_copy(table_hbm.at[idx_v.at[i, pl.ds(0, CH0)]],
                         rows_v.at[b, pl.ds(0, CH0)], gsem.at[b])
        pltpu.async_copy(table_hbm.at[idx_v.at[i, pl.ds(CH0, CH1)]],
                         rows_v.at[b, pl.ds(CH0, CH1)], gsem.at[b])

    def wait_gather(i, b):
        pltpu.make_async_copy(table_hbm.at[idx_v.at[i, pl.ds(0, CH0)]],
                              rows_v.at[b, pl.ds(0, CH0)], gsem.at[b]).wait()
        pltpu.make_async_copy(table_hbm.at[idx_v.at[i, pl.ds(CH0, CH1)]],
                              rows_v.at[b, pl.ds(CH0, CH1)], gsem.at[b]).wait()

    def store(i, b):
        pltpu.async_copy(rows_v.at[b], out_hbm.at[row_base + i], ssem.at[b])

    def wait_store(i, b):
        pltpu.make_async_copy(rows_v.at[b], out_hbm.at[row_base + i],
                              ssem.at[b]).wait()

    for b in range(NBUF):
        gather(b, b)

    def group(g, carry):
        base = g * NBUF
        for b in range(NBUF):
            wait_gather(base + b, b)
            store(base + b, b)
        for b in range(NBUF):
            wait_store(base + b, b)
            gather(base + NBUF + b, b)
        return carry

    lax.fori_loop(0, NG - 1, group, 0, unroll=False)

    last = (NG - 1) * NBUF
    for b in range(NBUF):
        wait_gather(last + b, b)
        store(last + b, b)
    for b in range(NBUF):
        wait_store(last + b, b)


def kernel(input, table):
    return _sc_embed(input.astype(jnp.int32), table)
